# baseline (speedup 1.0000x reference)
import functools

import jax
import jax.numpy as jnp
from jax import lax
from jax.experimental import pallas as pl
from jax.experimental.pallas import tpu as pltpu
from jax.experimental.pallas import tpu_sc as plsc

N_NODES = 10000
IN_DIM = 128
HID_DIM = 128
OUT_DIM = 64
N_EDGES = 320000
N_LABEL = 100000

NC, NS = 2, 16
NW = NC * NS
CHUNK = 128

N_PAD = 10240
ROWS_PER_SUB = N_PAD // NS

HALF_W = 72
CNT_COL = IN_DIM - HALF_W

E1_CPW = 157
E1_PAD = NS * E1_CPW * CHUNK

L_CPW = 25
L_PAD = NW * L_CPW * CHUNK

Z_W = 65

_MESH = plsc.VectorSubcoreMesh(core_axis_name="c", subcore_axis_name="s")
_SC_PARAMS = pltpu.CompilerParams(use_tc_tiling_on_sc=False,
                                  needs_layout_passes=False)


def _zero_rows(buf, width):
  z16 = jnp.zeros((16,), jnp.float32)
  ncol = (width + 15) // 16

  def zrow(r, carry):
    for c in range(ncol):
      buf[r, pl.ds(min(c * 16, width - 16), 16)] = z16
    return carry

  lax.fori_loop(0, CHUNK, zrow, 0)


def _make_segsum(width, cpw, split_cols, depth_g=2, depth_s=2):
  @functools.partial(
      pl.kernel,
      out_type=jax.ShapeDtypeStruct((NC, N_PAD, width), jnp.float32),
      mesh=_MESH,
      compiler_params=_SC_PARAMS,
      scratch_types=[
          pltpu.VMEM((cpw, CHUNK), jnp.int32),
          pltpu.VMEM((cpw, CHUNK), jnp.int32),
          pltpu.VMEM(((depth_g + depth_s) * CHUNK, width), jnp.float32),
          pltpu.VMEM_SHARED((N_PAD, width), jnp.float32),
          pltpu.SemaphoreType.DMA,
          pltpu.SemaphoreType.DMA,
      ],
  )
  def segsum(tab_hbm, src_hbm, dst_hbm, out_hbm,
             src_v, dst_v, rows2, acc_sh, gsem, ssem):
    cid = lax.axis_index("c")
    sid = lax.axis_index("s")
    r0 = sid * ROWS_PER_SUB
    tab = tab_hbm.at[cid] if split_cols else tab_hbm
    isel = sid if split_cols else cid * NS + sid

    _zero_rows(rows2, width)
    for t in range(ROWS_PER_SUB // CHUNK):
      pltpu.sync_copy(rows2.at[pl.ds(0, CHUNK)],
                      acc_sh.at[pl.ds(r0 + t * CHUNK, CHUNK)])
    plsc.subcore_barrier()
    pltpu.sync_copy(src_hbm.at[isel], src_v)
    pltpu.sync_copy(dst_hbm.at[isel], dst_v)

    nbuf = depth_g + depth_s

    def buf(v):
      return rows2.at[pl.ds((v % nbuf) * CHUNK, CHUNK)]

    def g_start(j, v):
      pltpu.async_copy(tab.at[src_v.at[j]], buf(v), gsem)

    def g_wait(v):
      pltpu.make_async_copy(tab.at[src_v.at[0]], buf(v), gsem).wait()

    def s_start(v):
      pltpu.async_copy(buf(v), acc_sh.at[dst_v.at[v]], ssem, add=True)

    def s_wait():
      pltpu.make_async_copy(buf(0), acc_sh.at[dst_v.at[0]], ssem).wait()

    def body(v, carry):
      u = v - depth_g

      @pl.when(v >= depth_g)
      def _process():
        g_wait(u)

        @pl.when(u >= depth_s)
        def _reclaim():
          s_wait()

        s_start(u)

      @pl.when(v < cpw)
      def _fetch():
        g_start(v, v)

      return carry

    lax.fori_loop(0, cpw + depth_g, body, 0)
    for _ in range(depth_s):
      s_wait()
    plsc.subcore_barrier()
    pltpu.sync_copy(acc_sh.at[pl.ds(r0, ROWS_PER_SUB)],
                    out_hbm.at[cid, pl.ds(r0, ROWS_PER_SUB)])

  return segsum


_segsum_l1 = _make_segsum(HALF_W, E1_CPW, split_cols=True)
_segsum_l2 = _make_segsum(OUT_DIM // 2, E1_CPW, split_cols=True,
                          depth_g=3, depth_s=2)


@functools.partial(
    pl.kernel,
    out_type=jax.ShapeDtypeStruct((L_PAD,), jnp.float32),
    mesh=_MESH,
    compiler_params=_SC_PARAMS,
    scratch_types=[
        pltpu.VMEM((L_CPW, CHUNK), jnp.int32),
        pltpu.VMEM((L_CPW, CHUNK), jnp.int32),
        pltpu.VMEM((4 * CHUNK, Z_W), jnp.float32),
        pltpu.VMEM((4 * CHUNK, Z_W), jnp.float32),
        pltpu.VMEM((L_CPW * CHUNK,), jnp.float32),
        pltpu.SemaphoreType.DMA,
        pltpu.SemaphoreType.DMA,
    ],
)
def _decode(z_hbm, ls_hbm, ld_hbm, out_hbm, ls_v, ld_v, zs2, zd2, out_v,
            ssm, dsm):
  cid = lax.axis_index("c")
  sid = lax.axis_index("s")
  wid = cid * NS + sid
  ib = wid * L_CPW
  pltpu.sync_copy(ls_hbm.at[wid], ls_v)
  pltpu.sync_copy(ld_hbm.at[wid], ld_v)

  def g_start(j, v):
    boff = (v % 4) * CHUNK
    pltpu.async_copy(z_hbm.at[ls_v.at[j]], zs2.at[pl.ds(boff, CHUNK)], ssm)
    pltpu.async_copy(z_hbm.at[ld_v.at[j]], zd2.at[pl.ds(boff, CHUNK)], dsm)

  def g_wait(v):
    boff = (v % 4) * CHUNK
    pltpu.make_async_copy(z_hbm.at[ls_v.at[0]], zs2.at[pl.ds(boff, CHUNK)],
                          ssm).wait()
    pltpu.make_async_copy(z_hbm.at[ld_v.at[0]], zd2.at[pl.ds(boff, CHUNK)],
                          dsm).wait()

  def visit(v, carry):
    u = v - 3

    @pl.when(v >= 3)
    def _process():
      g_wait(u)
      boff = (u % 4) * CHUNK
      lanes = lax.iota(jnp.int32, 16)

      def group(g, c2):
        rows = boff + g * 16 + lanes
        acc = jnp.zeros((16,), jnp.float32)
        for col in range(OUT_DIM):
          cv = jnp.full((16,), col, jnp.int32)
          acc = acc + (plsc.load_gather(zs2, [rows, cv]) *
                       plsc.load_gather(zd2, [rows, cv]))
        out_v[pl.ds(u * CHUNK + g * 16, 16)] = acc
        return c2

      lax.fori_loop(0, CHUNK // 16, group, 0)

    @pl.when(v < L_CPW)
    def _fetch():
      g_start(v, v)

    return carry

  lax.fori_loop(0, L_CPW + 3, visit, 0)
  pltpu.sync_copy(out_v, out_hbm.at[pl.ds(ib * CHUNK, L_CPW * CHUNK)])


def _layer1_body(aggp, xr, w1l, w1r, b1, w2l, w2r, b2, hw, hr, ic):
  a_lo = aggp[0]
  a_hi = aggp[1]
  inv = 1.0 / jnp.maximum(a_hi[:, CNT_COL], 1.0)
  m_lo = a_lo * inv[:, None]
  m_hi = a_hi[:, :CNT_COL] * inv[:, None]
  dn = (((1,), (1,)), ((), ()))
  f32 = jnp.float32
  h = (lax.dot_general(m_lo, w1l[:, :HALF_W], dn, preferred_element_type=f32)
       + lax.dot_general(m_hi, w1l[:, HALF_W:], dn, preferred_element_type=f32)
       + lax.dot_general(xr[...], w1r[...], dn, preferred_element_type=f32)
       + b1[...])
  h = jnp.maximum(h, 0.0)
  w2l_a = w2l[...]
  hw[0] = lax.dot_general(h, w2l_a[:OUT_DIM // 2], dn,
                          preferred_element_type=f32)
  hw[1] = lax.dot_general(h, w2l_a[OUT_DIM // 2:], dn,
                          preferred_element_type=f32)
  hr[...] = (lax.dot_general(h, w2r[...], dn, preferred_element_type=f32)
             + b2[...])
  ic[...] = inv[:, None]


def _layer1(aggp, x, W1_l, W1_r, b1, W2_l, W2_r, b2):
  BR = 1000
  return pl.pallas_call(
      lambda *refs: _layer1_body(refs[0][...], refs[1], refs[2][...],
                                 *refs[3:]),
      grid=(N_NODES // BR,),
      in_specs=[
          pl.BlockSpec((NC, BR, HALF_W), lambda i: (0, i, 0)),
          pl.BlockSpec((BR, IN_DIM), lambda i: (i, 0)),
          pl.BlockSpec((HID_DIM, IN_DIM), lambda i: (0, 0)),
          pl.BlockSpec((HID_DIM, IN_DIM), lambda i: (0, 0)),
          pl.BlockSpec((1, HID_DIM), lambda i: (0, 0)),
          pl.BlockSpec((OUT_DIM, HID_DIM), lambda i: (0, 0)),
          pl.BlockSpec((OUT_DIM, HID_DIM), lambda i: (0, 0)),
          pl.BlockSpec((1, OUT_DIM), lambda i: (0, 0)),
      ],
      out_specs=[
          pl.BlockSpec((NC, BR, OUT_DIM // 2), lambda i: (0, i, 0)),
          pl.BlockSpec((BR, OUT_DIM), lambda i: (i, 0)),
          pl.BlockSpec((BR, 1), lambda i: (i, 0)),
      ],
      out_shape=[
          jax.ShapeDtypeStruct((NC, N_NODES, OUT_DIM // 2), jnp.float32),
          jax.ShapeDtypeStruct((N_NODES, OUT_DIM), jnp.float32),
          jax.ShapeDtypeStruct((N_NODES, 1), jnp.float32),
      ],
  )(aggp, x, W1_l, W1_r, b1, W2_l, W2_r, b2)


def _assemble_body(aggp2, ic, hr, z):
  iv = ic[...]
  hra = hr[...]
  z[:, :OUT_DIM // 2] = aggp2[0] * iv + hra[:, :OUT_DIM // 2]
  z[:, OUT_DIM // 2:OUT_DIM] = aggp2[1] * iv + hra[:, OUT_DIM // 2:]
  z[:, OUT_DIM:] = jnp.zeros((z.shape[0], Z_W - OUT_DIM), jnp.float32)


def _assemble_z(aggp2, ic, hr):
  BR = 1000
  return pl.pallas_call(
      lambda *refs: _assemble_body(refs[0][...], *refs[1:]),
      grid=(N_NODES // BR,),
      in_specs=[
          pl.BlockSpec((NC, BR, OUT_DIM // 2), lambda i: (0, i, 0)),
          pl.BlockSpec((BR, 1), lambda i: (i, 0)),
          pl.BlockSpec((BR, OUT_DIM), lambda i: (i, 0)),
      ],
      out_specs=pl.BlockSpec((BR, Z_W), lambda i: (i, 0)),
      out_shape=jax.ShapeDtypeStruct((N_NODES, Z_W), jnp.float32),
  )(aggp2, ic, hr)


def kernel(x, edge_index, edge_label_index, W1_l, W1_r, b1, W2_l, W2_r, b2):
  i32 = jnp.int32
  f32 = jnp.float32
  src = edge_index[0].astype(i32)
  dst = edge_index[1].astype(i32)
  ls = edge_label_index[0].astype(i32)
  ld = edge_label_index[1].astype(i32)

  e1p = E1_PAD - N_EDGES
  src1 = jnp.concatenate([src, jnp.zeros((e1p,), i32)]).reshape(
      NS, E1_CPW, CHUNK)
  dst1 = jnp.concatenate([dst, jnp.full((e1p,), N_PAD - 1, i32)]).reshape(
      NS, E1_CPW, CHUNK)
  lp = L_PAD - N_LABEL
  ls2 = jnp.concatenate([ls, jnp.zeros((lp,), i32)]).reshape(NW, L_CPW, CHUNK)
  ld2 = jnp.concatenate([ld, jnp.zeros((lp,), i32)]).reshape(NW, L_CPW, CHUNK)

  xab = jnp.stack([
      x[:, :HALF_W],
      jnp.concatenate([x[:, HALF_W:], jnp.ones((N_NODES, 1), f32),
                       jnp.zeros((N_NODES, HALF_W - CNT_COL - 1), f32)],
                      axis=1),
  ])

  aggp1 = _segsum_l1(xab, src1, dst1)
  hw, hr, ic = _layer1(aggp1, x, W1_l, W1_r, b1.reshape(1, HID_DIM),
                       W2_l, W2_r, b2.reshape(1, OUT_DIM))
  aggp2 = _segsum_l2(hw, src1, dst1)
  z = _assemble_z(aggp2, ic, hr)
  out = _decode(z, ls2, ld2)
  return out[:N_LABEL]


# speedup vs baseline: 1.0574x; 1.0574x over previous
import functools

import jax
import jax.numpy as jnp
from jax import lax
from jax.experimental import pallas as pl
from jax.experimental.pallas import tpu as pltpu
from jax.experimental.pallas import tpu_sc as plsc

N_NODES = 10000
IN_DIM = 128
HID_DIM = 128
OUT_DIM = 64
N_EDGES = 320000
N_LABEL = 100000

NC, NS = 2, 16
NW = NC * NS
CHUNK = 128

N_PAD = 10240
ROWS_PER_SUB = N_PAD // NS

HALF_W = 72
CNT_COL = IN_DIM - HALF_W

E1_CPW = 157
E1_PAD = NS * E1_CPW * CHUNK

L_CPW = 25
L_PAD = NW * L_CPW * CHUNK

_MESH = plsc.VectorSubcoreMesh(core_axis_name="c", subcore_axis_name="s")
_SC_PARAMS = pltpu.CompilerParams(use_tc_tiling_on_sc=False,
                                  needs_layout_passes=False)


def _zero_rows(buf, width):
  z16 = jnp.zeros((16,), jnp.float32)
  ncol = (width + 15) // 16

  def zrow(r, carry):
    for c in range(ncol):
      buf[r, pl.ds(min(c * 16, width - 16), 16)] = z16
    return carry

  lax.fori_loop(0, CHUNK, zrow, 0)


def _make_segsum(width, cpw, split_cols, depth_g=2, depth_s=2):
  @functools.partial(
      pl.kernel,
      out_type=jax.ShapeDtypeStruct((NC, N_PAD, width), jnp.float32),
      mesh=_MESH,
      compiler_params=_SC_PARAMS,
      scratch_types=[
          pltpu.VMEM((cpw, CHUNK), jnp.int32),
          pltpu.VMEM((cpw, CHUNK), jnp.int32),
          pltpu.VMEM(((depth_g + depth_s) * CHUNK, width), jnp.float32),
          pltpu.VMEM_SHARED((N_PAD, width), jnp.float32),
          pltpu.SemaphoreType.DMA,
          pltpu.SemaphoreType.DMA,
      ],
  )
  def segsum(tab_hbm, src_hbm, dst_hbm, out_hbm,
             src_v, dst_v, rows2, acc_sh, gsem, ssem):
    cid = lax.axis_index("c")
    sid = lax.axis_index("s")
    r0 = sid * ROWS_PER_SUB
    tab = tab_hbm.at[cid] if split_cols else tab_hbm
    isel = sid if split_cols else cid * NS + sid

    _zero_rows(rows2, width)
    for t in range(ROWS_PER_SUB // CHUNK):
      pltpu.sync_copy(rows2.at[pl.ds(0, CHUNK)],
                      acc_sh.at[pl.ds(r0 + t * CHUNK, CHUNK)])
    plsc.subcore_barrier()
    pltpu.sync_copy(src_hbm.at[isel], src_v)
    pltpu.sync_copy(dst_hbm.at[isel], dst_v)

    nbuf = depth_g + depth_s

    def buf(v):
      return rows2.at[pl.ds((v % nbuf) * CHUNK, CHUNK)]

    def g_start(j, v):
      pltpu.async_copy(tab.at[src_v.at[j]], buf(v), gsem)

    def g_wait(v):
      pltpu.make_async_copy(tab.at[src_v.at[0]], buf(v), gsem).wait()

    def s_start(v):
      pltpu.async_copy(buf(v), acc_sh.at[dst_v.at[v]], ssem, add=True)

    def s_wait():
      pltpu.make_async_copy(buf(0), acc_sh.at[dst_v.at[0]], ssem).wait()

    def body(v, carry):
      u = v - depth_g

      @pl.when(v >= depth_g)
      def _process():
        g_wait(u)

        @pl.when(u >= depth_s)
        def _reclaim():
          s_wait()

        s_start(u)

      @pl.when(v < cpw)
      def _fetch():
        g_start(v, v)

      return carry

    lax.fori_loop(0, cpw + depth_g, body, 0)
    for _ in range(depth_s):
      s_wait()
    plsc.subcore_barrier()
    pltpu.sync_copy(acc_sh.at[pl.ds(r0, ROWS_PER_SUB)],
                    out_hbm.at[cid, pl.ds(r0, ROWS_PER_SUB)])

  return segsum


_segsum_l1 = _make_segsum(HALF_W, E1_CPW, split_cols=True)
_segsum_l2 = _make_segsum(OUT_DIM // 2, E1_CPW, split_cols=True,
                          depth_g=3, depth_s=2)


@functools.partial(
    pl.kernel,
    out_type=jax.ShapeDtypeStruct((L_PAD,), jnp.float32),
    mesh=_MESH,
    compiler_params=_SC_PARAMS,
    scratch_types=[
        pltpu.VMEM((L_CPW, CHUNK), jnp.int32),
        pltpu.VMEM((L_CPW, CHUNK), jnp.int32),
        pltpu.VMEM((4 * CHUNK, OUT_DIM), jnp.float32),
        pltpu.VMEM((4 * CHUNK, OUT_DIM), jnp.float32),
        pltpu.VMEM((L_CPW * CHUNK,), jnp.float32),
        pltpu.SemaphoreType.DMA,
        pltpu.SemaphoreType.DMA,
    ],
)
def _decode(z_hbm, ls_hbm, ld_hbm, out_hbm, ls_v, ld_v, zs2, zd2, out_v,
            ssm, dsm):
  cid = lax.axis_index("c")
  sid = lax.axis_index("s")
  wid = cid * NS + sid
  ib = wid * L_CPW
  pltpu.sync_copy(ls_hbm.at[wid], ls_v)
  pltpu.sync_copy(ld_hbm.at[wid], ld_v)

  def g_start(j, v):
    boff = (v % 4) * CHUNK
    pltpu.async_copy(z_hbm.at[ls_v.at[j]], zs2.at[pl.ds(boff, CHUNK)], ssm)
    pltpu.async_copy(z_hbm.at[ld_v.at[j]], zd2.at[pl.ds(boff, CHUNK)], dsm)

  def g_wait(v):
    boff = (v % 4) * CHUNK
    pltpu.make_async_copy(z_hbm.at[ls_v.at[0]], zs2.at[pl.ds(boff, CHUNK)],
                          ssm).wait()
    pltpu.make_async_copy(z_hbm.at[ld_v.at[0]], zd2.at[pl.ds(boff, CHUNK)],
                          dsm).wait()

  def visit(v, carry):
    u = v - 3

    @pl.when(v >= 3)
    def _process():
      g_wait(u)
      boff = (u % 4) * CHUNK
      lanes = lax.iota(jnp.int32, 16)

      def group(g, c2):
        r0 = boff + g * 16
        sv = jnp.zeros((16,), jnp.float32)
        for p in range(16):
          t = (zs2[r0 + p, pl.ds(0, 16)] * zd2[r0 + p, pl.ds(0, 16)])
          for c in range(1, OUT_DIM // 16):
            t = t + (zs2[r0 + p, pl.ds(c * 16, 16)] *
                     zd2[r0 + p, pl.ds(c * 16, 16)])
          s = jnp.sum(t)
          sv = jnp.where(lanes == p, s, sv)
        out_v[pl.ds(u * CHUNK + g * 16, 16)] = sv
        return c2

      lax.fori_loop(0, CHUNK // 16, group, 0)

    @pl.when(v < L_CPW)
    def _fetch():
      g_start(v, v)

    return carry

  lax.fori_loop(0, L_CPW + 3, visit, 0)
  pltpu.sync_copy(out_v, out_hbm.at[pl.ds(ib * CHUNK, L_CPW * CHUNK)])


def _layer1_body(aggp, xr, w1l, w1r, b1, w2l, w2r, b2, hw, hr, ic):
  a_lo = aggp[0]
  a_hi = aggp[1]
  inv = 1.0 / jnp.maximum(a_hi[:, CNT_COL], 1.0)
  m_lo = a_lo * inv[:, None]
  m_hi = a_hi[:, :CNT_COL] * inv[:, None]
  dn = (((1,), (1,)), ((), ()))
  f32 = jnp.float32
  h = (lax.dot_general(m_lo, w1l[:, :HALF_W], dn, preferred_element_type=f32)
       + lax.dot_general(m_hi, w1l[:, HALF_W:], dn, preferred_element_type=f32)
       + lax.dot_general(xr[...], w1r[...], dn, preferred_element_type=f32)
       + b1[...])
  h = jnp.maximum(h, 0.0)
  w2l_a = w2l[...]
  hw[0] = lax.dot_general(h, w2l_a[:OUT_DIM // 2], dn,
                          preferred_element_type=f32)
  hw[1] = lax.dot_general(h, w2l_a[OUT_DIM // 2:], dn,
                          preferred_element_type=f32)
  hr[...] = (lax.dot_general(h, w2r[...], dn, preferred_element_type=f32)
             + b2[...])
  ic[...] = inv[:, None]


def _layer1(aggp, x, W1_l, W1_r, b1, W2_l, W2_r, b2):
  BR = 1000
  return pl.pallas_call(
      lambda *refs: _layer1_body(refs[0][...], refs[1], refs[2][...],
                                 *refs[3:]),
      grid=(N_NODES // BR,),
      in_specs=[
          pl.BlockSpec((NC, BR, HALF_W), lambda i: (0, i, 0)),
          pl.BlockSpec((BR, IN_DIM), lambda i: (i, 0)),
          pl.BlockSpec((HID_DIM, IN_DIM), lambda i: (0, 0)),
          pl.BlockSpec((HID_DIM, IN_DIM), lambda i: (0, 0)),
          pl.BlockSpec((1, HID_DIM), lambda i: (0, 0)),
          pl.BlockSpec((OUT_DIM, HID_DIM), lambda i: (0, 0)),
          pl.BlockSpec((OUT_DIM, HID_DIM), lambda i: (0, 0)),
          pl.BlockSpec((1, OUT_DIM), lambda i: (0, 0)),
      ],
      out_specs=[
          pl.BlockSpec((NC, BR, OUT_DIM // 2), lambda i: (0, i, 0)),
          pl.BlockSpec((BR, OUT_DIM), lambda i: (i, 0)),
          pl.BlockSpec((BR, 1), lambda i: (i, 0)),
      ],
      out_shape=[
          jax.ShapeDtypeStruct((NC, N_NODES, OUT_DIM // 2), jnp.float32),
          jax.ShapeDtypeStruct((N_NODES, OUT_DIM), jnp.float32),
          jax.ShapeDtypeStruct((N_NODES, 1), jnp.float32),
      ],
  )(aggp, x, W1_l, W1_r, b1, W2_l, W2_r, b2)


def _assemble_body(aggp2, ic, hr, z):
  iv = ic[...]
  hra = hr[...]
  z[:, :OUT_DIM // 2] = aggp2[0] * iv + hra[:, :OUT_DIM // 2]
  z[:, OUT_DIM // 2:] = aggp2[1] * iv + hra[:, OUT_DIM // 2:]


def _assemble_z(aggp2, ic, hr):
  BR = 1000
  return pl.pallas_call(
      lambda *refs: _assemble_body(refs[0][...], *refs[1:]),
      grid=(N_NODES // BR,),
      in_specs=[
          pl.BlockSpec((NC, BR, OUT_DIM // 2), lambda i: (0, i, 0)),
          pl.BlockSpec((BR, 1), lambda i: (i, 0)),
          pl.BlockSpec((BR, OUT_DIM), lambda i: (i, 0)),
      ],
      out_specs=pl.BlockSpec((BR, OUT_DIM), lambda i: (i, 0)),
      out_shape=jax.ShapeDtypeStruct((N_NODES, OUT_DIM), jnp.float32),
  )(aggp2, ic, hr)


def kernel(x, edge_index, edge_label_index, W1_l, W1_r, b1, W2_l, W2_r, b2):
  i32 = jnp.int32
  f32 = jnp.float32
  src = edge_index[0].astype(i32)
  dst = edge_index[1].astype(i32)
  ls = edge_label_index[0].astype(i32)
  ld = edge_label_index[1].astype(i32)

  e1p = E1_PAD - N_EDGES
  src1 = jnp.concatenate([src, jnp.zeros((e1p,), i32)]).reshape(
      NS, E1_CPW, CHUNK)
  dst1 = jnp.concatenate([dst, jnp.full((e1p,), N_PAD - 1, i32)]).reshape(
      NS, E1_CPW, CHUNK)
  lp = L_PAD - N_LABEL
  ls2 = jnp.concatenate([ls, jnp.zeros((lp,), i32)]).reshape(NW, L_CPW, CHUNK)
  ld2 = jnp.concatenate([ld, jnp.zeros((lp,), i32)]).reshape(NW, L_CPW, CHUNK)

  xab = jnp.stack([
      x[:, :HALF_W],
      jnp.concatenate([x[:, HALF_W:], jnp.ones((N_NODES, 1), f32),
                       jnp.zeros((N_NODES, HALF_W - CNT_COL - 1), f32)],
                      axis=1),
  ])

  aggp1 = _segsum_l1(xab, src1, dst1)
  hw, hr, ic = _layer1(aggp1, x, W1_l, W1_r, b1.reshape(1, HID_DIM),
                       W2_l, W2_r, b2.reshape(1, OUT_DIM))
  aggp2 = _segsum_l2(hw, src1, dst1)
  z = _assemble_z(aggp2, ic, hr)
  out = _decode(z, ls2, ld2)
  return out[:N_LABEL]
